# Initial kernel scaffold; baseline (speedup 1.0000x reference)
import jax
import jax.numpy as jnp
from jax.experimental import pallas as pl


def kernel(inputs_s, inputs_sm, inputs_c, inputs_co, inputs_sl, edge_index_sim, edge_index_user, edge_sub_src, edge_sub_dst, params):
    raise NotImplementedError("write your pallas kernel here")


# speedup vs baseline: 1.5707x; 1.5707x over previous
import jax
import jax.numpy as jnp
from jax import lax
from jax.experimental import pallas as pl
from jax.experimental.pallas import tpu as pltpu
from jax.experimental.pallas import tpu_sc as plsc

N = 50000
E = 800000
L = 20
EMB = 16
HID = 64
NC = 2
NS = 16
CH = 128
ECHUNKS = E // CH
CORE_CHUNKS = ECHUNKS // NC
SUB_ITERS = (CORE_CHUNKS + NS - 1) // NS
CLS_ITERS = (ECHUNKS + NC * NS - 1) // (NC * NS)
DSEG = 1000
NSEG = N // DSEG
SEG_ITERS = (NSEG + NS - 1) // NS
CW = 16
NP = HID // CW
BN = 1000
BE = 4000
f32 = jnp.float32


def _mesh():
    return plsc.VectorSubcoreMesh(core_axis_name="c", subcore_axis_name="s",
                                  num_cores=NC, num_subcores=NS)


_SC_PARAMS = pltpu.CompilerParams(use_tc_tiling_on_sc=False)


def _deg_body(idx4, zeros8, ones8, out, idxv, onev, dv, sh0, sh1):
    c = lax.axis_index("c")
    s = lax.axis_index("s")
    shs = [sh0, sh1]
    pltpu.sync_copy(ones8, onev)
    base = c * CORE_CHUNKS
    for rnd in range(2):
        pltpu.sync_copy(zeros8, dv)
        for j in range(SEG_ITERS):
            k = s + NS * j

            @pl.when(k < NSEG)
            def _():
                for b in range(2):
                    pltpu.sync_copy(dv, shs[b].at[pl.ds(k * DSEG, DSEG)])
        plsc.subcore_barrier()

        def loop(i, carry):
            chunk = base + s + NS * i

            @pl.when(chunk < base + CORE_CHUNKS)
            def _():
                off = pl.multiple_of(chunk * CH, CH)
                for b in range(2):
                    pltpu.sync_copy(idx4.at[2 * rnd + b].at[pl.ds(off, CH)], idxv)
                    pltpu.sync_copy(onev, shs[b].at[idxv], add=True)
            return carry

        lax.fori_loop(0, SUB_ITERS, loop, 0)
        plsc.subcore_barrier()
        for j in range(SEG_ITERS):
            k = s + NS * j

            @pl.when(k < NSEG)
            def _():
                for b in range(2):
                    pltpu.sync_copy(shs[b].at[pl.ds(k * DSEG, DSEG)], dv)
                    pltpu.sync_copy(dv, out.at[c, 2 * rnd + b].at[pl.ds(k * DSEG, DSEG)])
        plsc.subcore_barrier()


def _sc_degrees(idx4, zeros8, ones8):
    return pl.kernel(
        _deg_body,
        out_type=jax.ShapeDtypeStruct((NC, 4, N, 8), f32),
        mesh=_mesh(),
        compiler_params=_SC_PARAMS,
        scratch_types=[
            pltpu.VMEM((CH,), jnp.int32),
            pltpu.VMEM((CH, 8), f32),
            pltpu.VMEM((DSEG, 8), f32),
            pltpu.VMEM_SHARED((N, 8), f32),
            pltpu.VMEM_SHARED((N, 8), f32),
        ],
    )(idx4, zeros8, ones8)


def _conv_body(hn0, hn1, hn2, hn3, srcd, zeros16, out, idxs, idxd, rows, zv,
               agg, sem):
    c = lax.axis_index("c")
    s = lax.axis_index("s")
    base = c * CORE_CHUNKS
    for p, hn in enumerate((hn0, hn1, hn2, hn3)):
        pltpu.sync_copy(zeros16, zv)
        for j in range(SEG_ITERS):
            k = s + NS * j

            @pl.when(k < NSEG)
            def _():
                pltpu.sync_copy(zv, agg.at[pl.ds(k * DSEG, DSEG)])
        plsc.subcore_barrier()

        def loop(i, carry):
            chunk = base + s + NS * i

            @pl.when(chunk < base + CORE_CHUNKS)
            def _():
                off = pl.multiple_of(chunk * CH, CH)
                pltpu.sync_copy(srcd.at[0].at[pl.ds(off, CH)], idxs)
                pltpu.async_copy(hn.at[idxs], rows, sem).wait()
                pltpu.sync_copy(srcd.at[1].at[pl.ds(off, CH)], idxd)
                pltpu.sync_copy(rows, agg.at[idxd], add=True)
            return carry

        lax.fori_loop(0, SUB_ITERS, loop, 0)
        plsc.subcore_barrier()
        for j in range(SEG_ITERS):
            k = s + NS * j

            @pl.when(k < NSEG)
            def _():
                pltpu.sync_copy(agg.at[pl.ds(k * DSEG, DSEG)], zv)
                pltpu.sync_copy(zv, out.at[c, p].at[pl.ds(k * DSEG, DSEG)])
        plsc.subcore_barrier()


def _sc_conv(hn, srcd, zeros16):
    return pl.kernel(
        _conv_body,
        out_type=jax.ShapeDtypeStruct((NC, NP, N, CW), f32),
        mesh=_mesh(),
        compiler_params=_SC_PARAMS,
        scratch_types=[
            pltpu.VMEM((CH,), jnp.int32),
            pltpu.VMEM((CH,), jnp.int32),
            pltpu.VMEM((CH, CW), f32),
            pltpu.VMEM((DSEG, CW), f32),
            pltpu.VMEM_SHARED((N, CW), f32),
            pltpu.SemaphoreType.DMA,
        ],
    )(hn[0], hn[1], hn[2], hn[3], srcd, zeros16)


def _cls_body(a_tab, b_tab, ss, sd, out, ia, ib, bufa, bufb, bufo, sem):
    c = lax.axis_index("c")
    s = lax.axis_index("s")
    w = s * NC + c

    def loop(i, carry):
        chunk = w + NC * NS * i

        @pl.when(chunk < ECHUNKS)
        def _():
            off = pl.multiple_of(chunk * CH, CH)
            pltpu.sync_copy(ss.at[pl.ds(off, CH)], ia)
            pltpu.async_copy(a_tab.at[ia], bufa, sem).wait()
            pltpu.sync_copy(sd.at[pl.ds(off, CH)], ib)
            pltpu.async_copy(b_tab.at[ib], bufb, sem).wait()

            def radd(r, c2):
                for j in range(4):
                    bufo[r, pl.ds(16 * j, 16)] = (
                        bufa[r, pl.ds(16 * j, 16)] + bufb[r, pl.ds(16 * j, 16)])
                return c2

            lax.fori_loop(0, CH, radd, 0)
            pltpu.sync_copy(bufo, out.at[pl.ds(off, CH)])
        return carry

    lax.fori_loop(0, CLS_ITERS, loop, 0)


def _sc_edge_sum(a_tab, b_tab, ss, sd):
    return pl.kernel(
        _cls_body,
        out_type=jax.ShapeDtypeStruct((E, HID), f32),
        mesh=_mesh(),
        compiler_params=_SC_PARAMS,
        scratch_types=[
            pltpu.VMEM((CH,), jnp.int32),
            pltpu.VMEM((CH,), jnp.int32),
            pltpu.VMEM((CH, HID), f32),
            pltpu.VMEM((CH, HID), f32),
            pltpu.VMEM((CH, HID), f32),
            pltpu.SemaphoreType.DMA,
        ],
    )(a_tab, b_tab, ss, sd)


def _dot(a, b):
    return jnp.dot(a, b, preferred_element_type=f32,
                   precision=lax.Precision.HIGHEST)


def _leaky(x):
    return jnp.where(x >= 0, x, 0.01 * x)


def _nodes_body(s_ref, ids3_ref, tab_ref, utf_ref, utb_ref, fcw_ref, fcb_ref,
                emb3_ref, degp_ref, *out_and_scratch):
    hs_refs = out_and_scratch[0:NP]
    hu_refs = out_and_scratch[NP:2 * NP]
    xg_ref = out_and_scratch[2 * NP]
    ids = s_ref[...]
    iot = lax.broadcasted_iota(jnp.int32, (1, 128), 1)
    tab = tab_ref[...]
    for t in range(L):
        oh = (ids[:, t:t + 1] == iot).astype(f32)
        xg_ref[t] = _dot(oh, tab)
    utf = utf_ref[...]
    utb = utb_ref[...]
    z = jnp.zeros((BN, EMB), f32)
    hf, cf, hb, cb = z, z, z, z
    for t in range(L):
        gf = xg_ref[t][:, :4 * EMB] + _dot(hf, utf)
        gb = xg_ref[L - 1 - t][:, 4 * EMB:] + _dot(hb, utb)
        i_f = jax.nn.sigmoid(gf[:, :EMB])
        f_f = jax.nn.sigmoid(gf[:, EMB:2 * EMB])
        g_f = jnp.tanh(gf[:, 2 * EMB:3 * EMB])
        o_f = jax.nn.sigmoid(gf[:, 3 * EMB:])
        cf = f_f * cf + i_f * g_f
        hf = o_f * jnp.tanh(cf)
        i_b = jax.nn.sigmoid(gb[:, :EMB])
        f_b = jax.nn.sigmoid(gb[:, EMB:2 * EMB])
        g_b = jnp.tanh(gb[:, 2 * EMB:3 * EMB])
        o_b = jax.nn.sigmoid(gb[:, 3 * EMB:])
        cb = f_b * cb + i_b * g_b
        hb = o_b * jnp.tanh(cb)
    h_url = _leaky(_dot(jnp.concatenate([hf, hb], 1), fcw_ref[...]) + fcb_ref[...])
    ids3 = ids3_ref[...]
    hc = _dot((ids3[:, 0:1] == iot).astype(f32), emb3_ref[0])
    hco = _dot((ids3[:, 1:2] == iot).astype(f32), emb3_ref[1])
    hsl = _dot((ids3[:, 2:3] == iot).astype(f32), emb3_ref[2])
    h = jnp.concatenate([h_url, hc, hco, hsl], 1)
    degp = degp_ref[...]

    def rfac(a):
        d = degp[0, a, :, 0:1] + degp[1, a, :, 0:1]
        return lax.rsqrt(jnp.clip(d, 1.0))

    hn_s = h * rfac(0)
    hn_u = h * rfac(2)
    for q in range(NP):
        hs_refs[q][...] = hn_s[:, CW * q:CW * (q + 1)]
        hu_refs[q][...] = hn_u[:, CW * q:CW * (q + 1)]


def _tc_nodes(inputs_s, ids3, tab, utf, utb, fcw, fcb, emb3, degp):
    g = N // BN
    quarter = jax.ShapeDtypeStruct((N, CW), f32)
    outs = pl.pallas_call(
        _nodes_body,
        grid=(g,),
        in_specs=[
            pl.BlockSpec((BN, L), lambda i: (i, 0)),
            pl.BlockSpec((BN, 4), lambda i: (i, 0)),
            pl.BlockSpec((128, 128), lambda i: (0, 0)),
            pl.BlockSpec((EMB, 4 * EMB), lambda i: (0, 0)),
            pl.BlockSpec((EMB, 4 * EMB), lambda i: (0, 0)),
            pl.BlockSpec((2 * EMB, EMB), lambda i: (0, 0)),
            pl.BlockSpec((1, EMB), lambda i: (0, 0)),
            pl.BlockSpec((3, 128, EMB), lambda i: (0, 0, 0)),
            pl.BlockSpec((NC, 4, BN, 8), lambda i: (0, 0, i, 0)),
        ],
        out_specs=[pl.BlockSpec((BN, CW), lambda i: (i, 0))] * (2 * NP),
        out_shape=[quarter] * (2 * NP),
        scratch_shapes=[pltpu.VMEM((L, BN, 128), f32)],
    )(inputs_s, ids3, tab, utf, utb, fcw, fcb, emb3, degp)
    return outs[0:NP], outs[NP:2 * NP]


def _combine_core(ps, pu, degp, ws, bs, wu, bu):
    def agg(pr, a):
        x = jnp.concatenate([pr[0, q] + pr[1, q] for q in range(NP)], 1)
        d = degp[0, a, :, 0:1] + degp[1, a, :, 0:1]
        return x * lax.rsqrt(jnp.clip(d, 1.0))

    hs = agg(ps, 1)
    hu = agg(pu, 3)
    return _leaky(_dot(hs, ws) + bs + _dot(hu, wu) + bu)


def _combine0_body(ps_ref, pu_ref, degp_ref, ws_ref, bs_ref, wu_ref, bu_ref,
                   *out_refs):
    degp = degp_ref[...]
    h = _combine_core(ps_ref[...], pu_ref[...], degp, ws_ref[...], bs_ref[...],
                      wu_ref[...], bu_ref[...])

    def rfac(a):
        d = degp[0, a, :, 0:1] + degp[1, a, :, 0:1]
        return lax.rsqrt(jnp.clip(d, 1.0))

    hn_s = h * rfac(0)
    hn_u = h * rfac(2)
    for q in range(NP):
        out_refs[q][...] = hn_s[:, CW * q:CW * (q + 1)]
        out_refs[NP + q][...] = hn_u[:, CW * q:CW * (q + 1)]


def _combine1_body(ps_ref, pu_ref, degp_ref, ws_ref, bs_ref, wu_ref, bu_ref,
                   w1a_ref, w1b_ref, kvec_ref, a_ref, b_ref):
    h = _combine_core(ps_ref[...], pu_ref[...], degp_ref[...], ws_ref[...],
                      bs_ref[...], wu_ref[...], bu_ref[...])
    a_ref[...] = _dot(h, w1a_ref[...]) + kvec_ref[...]
    b_ref[...] = _dot(h, w1b_ref[...])


_SPEC_PART = pl.BlockSpec((NC, NP, BN, CW), lambda i: (0, 0, i, 0))
_SPEC_DEGP = pl.BlockSpec((NC, 4, BN, 8), lambda i: (0, 0, i, 0))
_SPEC_W = pl.BlockSpec((HID, HID), lambda i: (0, 0))
_SPEC_B = pl.BlockSpec((1, HID), lambda i: (0, 0))


def _tc_combine0(ps, pu, degp, ws, bs, wu, bu):
    g = N // BN
    quarter = jax.ShapeDtypeStruct((N, CW), f32)
    outs = pl.pallas_call(
        _combine0_body,
        grid=(g,),
        in_specs=[_SPEC_PART, _SPEC_PART, _SPEC_DEGP,
                  _SPEC_W, _SPEC_B, _SPEC_W, _SPEC_B],
        out_specs=[pl.BlockSpec((BN, CW), lambda i: (i, 0))] * (2 * NP),
        out_shape=[quarter] * (2 * NP),
    )(ps, pu, degp, ws, bs, wu, bu)
    return outs[0:NP], outs[NP:2 * NP]


def _tc_combine1(ps, pu, degp, ws, bs, wu, bu, w1a, w1b, kvec):
    g = N // BN
    full = jax.ShapeDtypeStruct((N, HID), f32)
    return pl.pallas_call(
        _combine1_body,
        grid=(g,),
        in_specs=[_SPEC_PART, _SPEC_PART, _SPEC_DEGP,
                  _SPEC_W, _SPEC_B, _SPEC_W, _SPEC_B,
                  _SPEC_W, _SPEC_W, _SPEC_B],
        out_specs=[pl.BlockSpec((BN, HID), lambda i: (i, 0))] * 2,
        out_shape=[full, full],
    )(ps, pu, degp, ws, bs, wu, bu, w1a, w1b, kvec)


def _logits_body(s_ref, w2_ref, b2_ref, o_ref):
    x = jnp.maximum(s_ref[...], 0.0)
    o_ref[...] = _dot(x, w2_ref[...]) + b2_ref[...]


def _tc_logits(s_arr, w2p, b2p):
    g = E // BE
    return pl.pallas_call(
        _logits_body,
        grid=(g,),
        in_specs=[
            pl.BlockSpec((BE, HID), lambda i: (i, 0)),
            pl.BlockSpec((HID, 8), lambda i: (0, 0)),
            pl.BlockSpec((1, 8), lambda i: (0, 0)),
        ],
        out_specs=pl.BlockSpec((BE, 8), lambda i: (i, 0)),
        out_shape=jax.ShapeDtypeStruct((E, 8), f32),
    )(s_arr, w2p, b2p)


def kernel(inputs_s, inputs_sm, inputs_c, inputs_co, inputs_sl,
           edge_index_sim, edge_index_user, edge_sub_src, edge_sub_dst, params):
    p = params

    tab_f = p['emb_url'] @ p['w_ih_f'].T + p['b_ih_f'] + p['b_hh_f']
    tab_b = p['emb_url'] @ p['w_ih_b'].T + p['b_ih_b'] + p['b_hh_b']
    tab = jnp.concatenate([tab_f, tab_b], axis=1).astype(f32)
    utf = p['w_hh_f'].T.astype(f32)
    utb = p['w_hh_b'].T.astype(f32)
    fcw = p['fc_w'].astype(f32)
    fcb = p['fc_b'].reshape(1, EMB).astype(f32)

    def pad128(t):
        return jnp.zeros((128, EMB), f32).at[:t.shape[0]].set(t)

    emb3 = jnp.stack([pad128(p['emb_cat']), pad128(p['emb_country']),
                      pad128(p['emb_sl'])])

    sbn = p['bn_g'] * lax.rsqrt(p['bn_v'] + 1e-5)
    w1a = (p['cls_w1'][:HID] * sbn[None, :]).astype(f32)
    w1b = (p['cls_w1'][HID:] * sbn[None, :]).astype(f32)
    kvec = ((p['cls_b1'] - p['bn_m']) * sbn + p['bn_b']).reshape(1, HID).astype(f32)
    w2p = jnp.zeros((HID, 8), f32).at[:, :2].set(p['cls_w2'])
    b2p = jnp.zeros((1, 8), f32).at[:, :2].set(p['cls_b2'])

    ids3 = jnp.concatenate([inputs_c, inputs_co, inputs_sl, inputs_sl], axis=1)
    idx4 = jnp.stack([edge_index_sim[0], edge_index_sim[1],
                      edge_index_user[0], edge_index_user[1]])
    zeros8 = jnp.zeros((DSEG, 8), f32)
    ones8 = jnp.ones((CH, 8), f32)
    zeros16 = jnp.zeros((DSEG, CW), f32)

    degp = _sc_degrees(idx4, zeros8, ones8)
    hs, hu = _tc_nodes(inputs_s, ids3, tab, utf, utb, fcw, fcb, emb3, degp)

    ps = _sc_conv(hs, edge_index_sim, zeros16)
    pu = _sc_conv(hu, edge_index_user, zeros16)
    hs, hu = _tc_combine0(
        ps, pu, degp,
        p['gc0_sim_w'].astype(f32), p['gc0_sim_b'].reshape(1, HID).astype(f32),
        p['gc0_user_w'].astype(f32), p['gc0_user_b'].reshape(1, HID).astype(f32))

    ps = _sc_conv(hs, edge_index_sim, zeros16)
    pu = _sc_conv(hu, edge_index_user, zeros16)
    a_tab, b_tab = _tc_combine1(
        ps, pu, degp,
        p['gc1_sim_w'].astype(f32), p['gc1_sim_b'].reshape(1, HID).astype(f32),
        p['gc1_user_w'].astype(f32), p['gc1_user_b'].reshape(1, HID).astype(f32),
        w1a, w1b, kvec)

    s_arr = _sc_edge_sum(a_tab, b_tab, edge_sub_src, edge_sub_dst)
    out8 = _tc_logits(s_arr, w2p, b2p)
    return out8[:, :2]


# speedup vs baseline: 2.0854x; 1.3277x over previous
import jax
import jax.numpy as jnp
from jax import lax
from jax.experimental import pallas as pl
from jax.experimental.pallas import tpu as pltpu
from jax.experimental.pallas import tpu_sc as plsc

N = 50000
E = 800000
L = 20
EMB = 16
HID = 64
NC = 2
NS = 16
CH = 128
ECHUNKS = E // CH
CORE_CHUNKS = ECHUNKS // NC
SUB_ITERS = (CORE_CHUNKS + NS - 1) // NS
CLS_ITERS = (ECHUNKS + NC * NS - 1) // (NC * NS)
DSEG = 1000
NSEG = N // DSEG
SEG_ITERS = (NSEG + NS - 1) // NS
CW = 16
NP = HID // CW
CCH = 400
CECHUNKS = E // CCH
CCORE_CHUNKS = CECHUNKS // NC
CSUB_ITERS = (CCORE_CHUNKS + NS - 1) // NS
BN = 1000
BE = 4000
f32 = jnp.float32


def _mesh():
    return plsc.VectorSubcoreMesh(core_axis_name="c", subcore_axis_name="s",
                                  num_cores=NC, num_subcores=NS)


_SC_PARAMS = pltpu.CompilerParams(use_tc_tiling_on_sc=False)


def _deg_body(idx4, zeros8, ones8, out, idxv, onev, dv, sh0, sh1):
    c = lax.axis_index("c")
    s = lax.axis_index("s")
    shs = [sh0, sh1]
    pltpu.sync_copy(ones8, onev)
    base = c * CORE_CHUNKS
    for rnd in range(2):
        pltpu.sync_copy(zeros8, dv)
        for j in range(SEG_ITERS):
            k = s + NS * j

            @pl.when(k < NSEG)
            def _():
                for b in range(2):
                    pltpu.sync_copy(dv, shs[b].at[pl.ds(k * DSEG, DSEG)])
        plsc.subcore_barrier()

        def loop(i, carry):
            chunk = base + s + NS * i

            @pl.when(chunk < base + CORE_CHUNKS)
            def _():
                off = pl.multiple_of(chunk * CH, CH)
                for b in range(2):
                    pltpu.sync_copy(idx4.at[2 * rnd + b].at[pl.ds(off, CH)], idxv)
                    pltpu.sync_copy(onev, shs[b].at[idxv], add=True)
            return carry

        lax.fori_loop(0, SUB_ITERS, loop, 0)
        plsc.subcore_barrier()
        for j in range(SEG_ITERS):
            k = s + NS * j

            @pl.when(k < NSEG)
            def _():
                for b in range(2):
                    pltpu.sync_copy(shs[b].at[pl.ds(k * DSEG, DSEG)], dv)
                    pltpu.sync_copy(dv, out.at[c, 2 * rnd + b].at[pl.ds(k * DSEG, DSEG)])
        plsc.subcore_barrier()


def _sc_degrees(idx4, zeros8, ones8):
    return pl.kernel(
        _deg_body,
        out_type=jax.ShapeDtypeStruct((NC, 4, N, 8), f32),
        mesh=_mesh(),
        compiler_params=_SC_PARAMS,
        scratch_types=[
            pltpu.VMEM((CH,), jnp.int32),
            pltpu.VMEM((CH, 8), f32),
            pltpu.VMEM((DSEG, 8), f32),
            pltpu.VMEM_SHARED((N, 8), f32),
            pltpu.VMEM_SHARED((N, 8), f32),
        ],
    )(idx4, zeros8, ones8)


def _conv_body(hn0, hn1, hn2, hn3, srcd, zeros16, out, idxs, idxd, rows, zv,
               agg, sem):
    c = lax.axis_index("c")
    s = lax.axis_index("s")
    base = c * CCORE_CHUNKS
    for p, hn in enumerate((hn0, hn1, hn2, hn3)):
        pltpu.sync_copy(zeros16, zv)
        for j in range(SEG_ITERS):
            k = s + NS * j

            @pl.when(k < NSEG)
            def _():
                pltpu.sync_copy(zv, agg.at[pl.ds(k * DSEG, DSEG)])
        plsc.subcore_barrier()

        def loop(i, carry):
            chunk = base + s + NS * i

            @pl.when(chunk < base + CCORE_CHUNKS)
            def _():
                off = pl.multiple_of(chunk * CCH, CCH)
                pltpu.sync_copy(srcd.at[0].at[pl.ds(off, CCH)], idxs)
                pltpu.async_copy(hn.at[idxs], rows, sem).wait()
                pltpu.sync_copy(srcd.at[1].at[pl.ds(off, CCH)], idxd)
                pltpu.sync_copy(rows, agg.at[idxd], add=True)
            return carry

        lax.fori_loop(0, CSUB_ITERS, loop, 0)
        plsc.subcore_barrier()
        for j in range(SEG_ITERS):
            k = s + NS * j

            @pl.when(k < NSEG)
            def _():
                pltpu.sync_copy(agg.at[pl.ds(k * DSEG, DSEG)], zv)
                pltpu.sync_copy(zv, out.at[c, p].at[pl.ds(k * DSEG, DSEG)])
        plsc.subcore_barrier()


def _sc_conv(hn, srcd, zeros16):
    return pl.kernel(
        _conv_body,
        out_type=jax.ShapeDtypeStruct((NC, NP, N, CW), f32),
        mesh=_mesh(),
        compiler_params=_SC_PARAMS,
        scratch_types=[
            pltpu.VMEM((CCH,), jnp.int32),
            pltpu.VMEM((CCH,), jnp.int32),
            pltpu.VMEM((CCH, CW), f32),
            pltpu.VMEM((DSEG, CW), f32),
            pltpu.VMEM_SHARED((N, CW), f32),
            pltpu.SemaphoreType.DMA,
        ],
    )(hn[0], hn[1], hn[2], hn[3], srcd, zeros16)


def _cls_body(a_tab, b_tab, ss, sd, out, ia, ib, bufa, bufb, bufo, sem):
    c = lax.axis_index("c")
    s = lax.axis_index("s")
    w = s * NC + c

    def loop(i, carry):
        chunk = w + NC * NS * i

        @pl.when(chunk < ECHUNKS)
        def _():
            off = pl.multiple_of(chunk * CH, CH)
            pltpu.sync_copy(ss.at[pl.ds(off, CH)], ia)
            pltpu.async_copy(a_tab.at[ia], bufa, sem).wait()
            pltpu.sync_copy(sd.at[pl.ds(off, CH)], ib)
            pltpu.async_copy(b_tab.at[ib], bufb, sem).wait()

            def radd(r, c2):
                for j in range(4):
                    bufo[r, pl.ds(16 * j, 16)] = (
                        bufa[r, pl.ds(16 * j, 16)] + bufb[r, pl.ds(16 * j, 16)])
                return c2

            lax.fori_loop(0, CH, radd, 0)
            pltpu.sync_copy(bufo, out.at[pl.ds(off, CH)])
        return carry

    lax.fori_loop(0, CLS_ITERS, loop, 0)


def _sc_edge_sum(a_tab, b_tab, ss, sd):
    return pl.kernel(
        _cls_body,
        out_type=jax.ShapeDtypeStruct((E, HID), f32),
        mesh=_mesh(),
        compiler_params=_SC_PARAMS,
        scratch_types=[
            pltpu.VMEM((CH,), jnp.int32),
            pltpu.VMEM((CH,), jnp.int32),
            pltpu.VMEM((CH, HID), f32),
            pltpu.VMEM((CH, HID), f32),
            pltpu.VMEM((CH, HID), f32),
            pltpu.SemaphoreType.DMA,
        ],
    )(a_tab, b_tab, ss, sd)


def _dot(a, b):
    return jnp.dot(a, b, preferred_element_type=f32,
                   precision=lax.Precision.HIGHEST)


def _leaky(x):
    return jnp.where(x >= 0, x, 0.01 * x)


def _nodes_body(s_ref, ids3_ref, tab_ref, utf_ref, utb_ref, fcw_ref, fcb_ref,
                emb3_ref, degp_ref, *out_and_scratch):
    hs_refs = out_and_scratch[0:NP]
    hu_refs = out_and_scratch[NP:2 * NP]
    xg_ref = out_and_scratch[2 * NP]
    ids = s_ref[...]
    iot = lax.broadcasted_iota(jnp.int32, (1, 128), 1)
    tab = tab_ref[...]
    for t in range(L):
        oh = (ids[:, t:t + 1] == iot).astype(f32)
        xg_ref[t] = _dot(oh, tab)
    utf = utf_ref[...]
    utb = utb_ref[...]
    z = jnp.zeros((BN, EMB), f32)
    hf, cf, hb, cb = z, z, z, z
    for t in range(L):
        gf = xg_ref[t][:, :4 * EMB] + _dot(hf, utf)
        gb = xg_ref[L - 1 - t][:, 4 * EMB:] + _dot(hb, utb)
        i_f = jax.nn.sigmoid(gf[:, :EMB])
        f_f = jax.nn.sigmoid(gf[:, EMB:2 * EMB])
        g_f = jnp.tanh(gf[:, 2 * EMB:3 * EMB])
        o_f = jax.nn.sigmoid(gf[:, 3 * EMB:])
        cf = f_f * cf + i_f * g_f
        hf = o_f * jnp.tanh(cf)
        i_b = jax.nn.sigmoid(gb[:, :EMB])
        f_b = jax.nn.sigmoid(gb[:, EMB:2 * EMB])
        g_b = jnp.tanh(gb[:, 2 * EMB:3 * EMB])
        o_b = jax.nn.sigmoid(gb[:, 3 * EMB:])
        cb = f_b * cb + i_b * g_b
        hb = o_b * jnp.tanh(cb)
    h_url = _leaky(_dot(jnp.concatenate([hf, hb], 1), fcw_ref[...]) + fcb_ref[...])
    ids3 = ids3_ref[...]
    hc = _dot((ids3[:, 0:1] == iot).astype(f32), emb3_ref[0])
    hco = _dot((ids3[:, 1:2] == iot).astype(f32), emb3_ref[1])
    hsl = _dot((ids3[:, 2:3] == iot).astype(f32), emb3_ref[2])
    h = jnp.concatenate([h_url, hc, hco, hsl], 1)
    degp = degp_ref[...]

    def rfac(a):
        d = degp[0, a, :, 0:1] + degp[1, a, :, 0:1]
        return lax.rsqrt(jnp.clip(d, 1.0))

    hn_s = h * rfac(0)
    hn_u = h * rfac(2)
    for q in range(NP):
        hs_refs[q][...] = hn_s[:, CW * q:CW * (q + 1)]
        hu_refs[q][...] = hn_u[:, CW * q:CW * (q + 1)]


def _tc_nodes(inputs_s, ids3, tab, utf, utb, fcw, fcb, emb3, degp):
    g = N // BN
    quarter = jax.ShapeDtypeStruct((N, CW), f32)
    outs = pl.pallas_call(
        _nodes_body,
        grid=(g,),
        in_specs=[
            pl.BlockSpec((BN, L), lambda i: (i, 0)),
            pl.BlockSpec((BN, 4), lambda i: (i, 0)),
            pl.BlockSpec((128, 128), lambda i: (0, 0)),
            pl.BlockSpec((EMB, 4 * EMB), lambda i: (0, 0)),
            pl.BlockSpec((EMB, 4 * EMB), lambda i: (0, 0)),
            pl.BlockSpec((2 * EMB, EMB), lambda i: (0, 0)),
            pl.BlockSpec((1, EMB), lambda i: (0, 0)),
            pl.BlockSpec((3, 128, EMB), lambda i: (0, 0, 0)),
            pl.BlockSpec((NC, 4, BN, 8), lambda i: (0, 0, i, 0)),
        ],
        out_specs=[pl.BlockSpec((BN, CW), lambda i: (i, 0))] * (2 * NP),
        out_shape=[quarter] * (2 * NP),
        scratch_shapes=[pltpu.VMEM((L, BN, 128), f32)],
    )(inputs_s, ids3, tab, utf, utb, fcw, fcb, emb3, degp)
    return outs[0:NP], outs[NP:2 * NP]


def _combine_core(ps, pu, degp, ws, bs, wu, bu):
    def agg(pr, a):
        x = jnp.concatenate([pr[0, q] + pr[1, q] for q in range(NP)], 1)
        d = degp[0, a, :, 0:1] + degp[1, a, :, 0:1]
        return x * lax.rsqrt(jnp.clip(d, 1.0))

    hs = agg(ps, 1)
    hu = agg(pu, 3)
    return _leaky(_dot(hs, ws) + bs + _dot(hu, wu) + bu)


def _combine0_body(ps_ref, pu_ref, degp_ref, ws_ref, bs_ref, wu_ref, bu_ref,
                   *out_refs):
    degp = degp_ref[...]
    h = _combine_core(ps_ref[...], pu_ref[...], degp, ws_ref[...], bs_ref[...],
                      wu_ref[...], bu_ref[...])

    def rfac(a):
        d = degp[0, a, :, 0:1] + degp[1, a, :, 0:1]
        return lax.rsqrt(jnp.clip(d, 1.0))

    hn_s = h * rfac(0)
    hn_u = h * rfac(2)
    for q in range(NP):
        out_refs[q][...] = hn_s[:, CW * q:CW * (q + 1)]
        out_refs[NP + q][...] = hn_u[:, CW * q:CW * (q + 1)]


def _combine1_body(ps_ref, pu_ref, degp_ref, ws_ref, bs_ref, wu_ref, bu_ref,
                   w1a_ref, w1b_ref, kvec_ref, a_ref, b_ref):
    h = _combine_core(ps_ref[...], pu_ref[...], degp_ref[...], ws_ref[...],
                      bs_ref[...], wu_ref[...], bu_ref[...])
    a_ref[...] = _dot(h, w1a_ref[...]) + kvec_ref[...]
    b_ref[...] = _dot(h, w1b_ref[...])


_SPEC_PART = pl.BlockSpec((NC, NP, BN, CW), lambda i: (0, 0, i, 0))
_SPEC_DEGP = pl.BlockSpec((NC, 4, BN, 8), lambda i: (0, 0, i, 0))
_SPEC_W = pl.BlockSpec((HID, HID), lambda i: (0, 0))
_SPEC_B = pl.BlockSpec((1, HID), lambda i: (0, 0))


def _tc_combine0(ps, pu, degp, ws, bs, wu, bu):
    g = N // BN
    quarter = jax.ShapeDtypeStruct((N, CW), f32)
    outs = pl.pallas_call(
        _combine0_body,
        grid=(g,),
        in_specs=[_SPEC_PART, _SPEC_PART, _SPEC_DEGP,
                  _SPEC_W, _SPEC_B, _SPEC_W, _SPEC_B],
        out_specs=[pl.BlockSpec((BN, CW), lambda i: (i, 0))] * (2 * NP),
        out_shape=[quarter] * (2 * NP),
    )(ps, pu, degp, ws, bs, wu, bu)
    return outs[0:NP], outs[NP:2 * NP]


def _tc_combine1(ps, pu, degp, ws, bs, wu, bu, w1a, w1b, kvec):
    g = N // BN
    full = jax.ShapeDtypeStruct((N, HID), f32)
    return pl.pallas_call(
        _combine1_body,
        grid=(g,),
        in_specs=[_SPEC_PART, _SPEC_PART, _SPEC_DEGP,
                  _SPEC_W, _SPEC_B, _SPEC_W, _SPEC_B,
                  _SPEC_W, _SPEC_W, _SPEC_B],
        out_specs=[pl.BlockSpec((BN, HID), lambda i: (i, 0))] * 2,
        out_shape=[full, full],
    )(ps, pu, degp, ws, bs, wu, bu, w1a, w1b, kvec)


def _logits_body(s_ref, w2_ref, b2_ref, o_ref):
    x = jnp.maximum(s_ref[...], 0.0)
    o_ref[...] = _dot(x, w2_ref[...]) + b2_ref[...]


def _tc_logits(s_arr, w2p, b2p):
    g = E // BE
    return pl.pallas_call(
        _logits_body,
        grid=(g,),
        in_specs=[
            pl.BlockSpec((BE, HID), lambda i: (i, 0)),
            pl.BlockSpec((HID, 8), lambda i: (0, 0)),
            pl.BlockSpec((1, 8), lambda i: (0, 0)),
        ],
        out_specs=pl.BlockSpec((BE, 8), lambda i: (i, 0)),
        out_shape=jax.ShapeDtypeStruct((E, 8), f32),
    )(s_arr, w2p, b2p)


def kernel(inputs_s, inputs_sm, inputs_c, inputs_co, inputs_sl,
           edge_index_sim, edge_index_user, edge_sub_src, edge_sub_dst, params):
    p = params

    tab_f = p['emb_url'] @ p['w_ih_f'].T + p['b_ih_f'] + p['b_hh_f']
    tab_b = p['emb_url'] @ p['w_ih_b'].T + p['b_ih_b'] + p['b_hh_b']
    tab = jnp.concatenate([tab_f, tab_b], axis=1).astype(f32)
    utf = p['w_hh_f'].T.astype(f32)
    utb = p['w_hh_b'].T.astype(f32)
    fcw = p['fc_w'].astype(f32)
    fcb = p['fc_b'].reshape(1, EMB).astype(f32)

    def pad128(t):
        return jnp.zeros((128, EMB), f32).at[:t.shape[0]].set(t)

    emb3 = jnp.stack([pad128(p['emb_cat']), pad128(p['emb_country']),
                      pad128(p['emb_sl'])])

    sbn = p['bn_g'] * lax.rsqrt(p['bn_v'] + 1e-5)
    w1a = (p['cls_w1'][:HID] * sbn[None, :]).astype(f32)
    w1b = (p['cls_w1'][HID:] * sbn[None, :]).astype(f32)
    kvec = ((p['cls_b1'] - p['bn_m']) * sbn + p['bn_b']).reshape(1, HID).astype(f32)
    w2p = jnp.zeros((HID, 8), f32).at[:, :2].set(p['cls_w2'])
    b2p = jnp.zeros((1, 8), f32).at[:, :2].set(p['cls_b2'])

    ids3 = jnp.concatenate([inputs_c, inputs_co, inputs_sl, inputs_sl], axis=1)
    idx4 = jnp.stack([edge_index_sim[0], edge_index_sim[1],
                      edge_index_user[0], edge_index_user[1]])
    zeros8 = jnp.zeros((DSEG, 8), f32)
    ones8 = jnp.ones((CH, 8), f32)
    zeros16 = jnp.zeros((DSEG, CW), f32)

    degp = _sc_degrees(idx4, zeros8, ones8)
    hs, hu = _tc_nodes(inputs_s, ids3, tab, utf, utb, fcw, fcb, emb3, degp)

    ps = _sc_conv(hs, edge_index_sim, zeros16)
    pu = _sc_conv(hu, edge_index_user, zeros16)
    hs, hu = _tc_combine0(
        ps, pu, degp,
        p['gc0_sim_w'].astype(f32), p['gc0_sim_b'].reshape(1, HID).astype(f32),
        p['gc0_user_w'].astype(f32), p['gc0_user_b'].reshape(1, HID).astype(f32))

    ps = _sc_conv(hs, edge_index_sim, zeros16)
    pu = _sc_conv(hu, edge_index_user, zeros16)
    a_tab, b_tab = _tc_combine1(
        ps, pu, degp,
        p['gc1_sim_w'].astype(f32), p['gc1_sim_b'].reshape(1, HID).astype(f32),
        p['gc1_user_w'].astype(f32), p['gc1_user_b'].reshape(1, HID).astype(f32),
        w1a, w1b, kvec)

    s_arr = _sc_edge_sum(a_tab, b_tab, edge_sub_src, edge_sub_dst)
    out8 = _tc_logits(s_arr, w2p, b2p)
    return out8[:, :2]


# speedup vs baseline: 2.8568x; 1.3699x over previous
import jax
import jax.numpy as jnp
from jax import lax
from jax.experimental import pallas as pl
from jax.experimental.pallas import tpu as pltpu
from jax.experimental.pallas import tpu_sc as plsc

N = 50000
E = 800000
L = 20
EMB = 16
HID = 64
NC = 2
NS = 16
CH = 128
ECHUNKS = E // CH
CORE_CHUNKS = ECHUNKS // NC
SUB_ITERS = (CORE_CHUNKS + NS - 1) // NS
CLS_ITERS = (ECHUNKS + NC * NS - 1) // (NC * NS)
DSEG = 1000
NSEG = N // DSEG
SEG_ITERS = (NSEG + NS - 1) // NS
CW = 16
NP = HID // CW
CCH = 400
CECHUNKS = E // CCH
CCORE_CHUNKS = CECHUNKS // NC
CSUB_ITERS = (CCORE_CHUNKS + NS - 1) // NS
BN = 1000
BE = 4000
f32 = jnp.float32


def _mesh():
    return plsc.VectorSubcoreMesh(core_axis_name="c", subcore_axis_name="s",
                                  num_cores=NC, num_subcores=NS)


_SC_PARAMS = pltpu.CompilerParams(use_tc_tiling_on_sc=False)


def _deg_body(idx4, zeros8, ones8, out, idxv, onev, dv, sh0, sh1):
    c = lax.axis_index("c")
    s = lax.axis_index("s")
    shs = [sh0, sh1]
    pltpu.sync_copy(ones8, onev)
    base = c * CORE_CHUNKS
    for rnd in range(2):
        pltpu.sync_copy(zeros8, dv)
        for j in range(SEG_ITERS):
            k = s + NS * j

            @pl.when(k < NSEG)
            def _():
                for b in range(2):
                    pltpu.sync_copy(dv, shs[b].at[pl.ds(k * DSEG, DSEG)])
        plsc.subcore_barrier()

        def loop(i, carry):
            chunk = base + s + NS * i

            @pl.when(chunk < base + CORE_CHUNKS)
            def _():
                off = pl.multiple_of(chunk * CH, CH)
                for b in range(2):
                    pltpu.sync_copy(idx4.at[2 * rnd + b].at[pl.ds(off, CH)], idxv)
                    pltpu.sync_copy(onev, shs[b].at[idxv], add=True)
            return carry

        lax.fori_loop(0, SUB_ITERS, loop, 0)
        plsc.subcore_barrier()
        for j in range(SEG_ITERS):
            k = s + NS * j

            @pl.when(k < NSEG)
            def _():
                for b in range(2):
                    pltpu.sync_copy(shs[b].at[pl.ds(k * DSEG, DSEG)], dv)
                    pltpu.sync_copy(dv, out.at[c, 2 * rnd + b].at[pl.ds(k * DSEG, DSEG)])
        plsc.subcore_barrier()


def _sc_degrees(idx4, zeros8, ones8):
    return pl.kernel(
        _deg_body,
        out_type=jax.ShapeDtypeStruct((NC, 4, N, 8), f32),
        mesh=_mesh(),
        compiler_params=_SC_PARAMS,
        scratch_types=[
            pltpu.VMEM((CH,), jnp.int32),
            pltpu.VMEM((CH, 8), f32),
            pltpu.VMEM((DSEG, 8), f32),
            pltpu.VMEM_SHARED((N, 8), f32),
            pltpu.VMEM_SHARED((N, 8), f32),
        ],
    )(idx4, zeros8, ones8)


def _conv_body(hn0, hn1, hn2, hn3, srcd, zeros16, out, idxs, idxd, rows, zv,
               agg, sem):
    c = lax.axis_index("c")
    s = lax.axis_index("s")
    base = c * CCORE_CHUNKS
    for p, hn in enumerate((hn0, hn1, hn2, hn3)):
        pltpu.sync_copy(zeros16, zv)
        for j in range(SEG_ITERS):
            k = s + NS * j

            @pl.when(k < NSEG)
            def _():
                pltpu.sync_copy(zv, agg.at[pl.ds(k * DSEG, DSEG)])
        plsc.subcore_barrier()

        def loop(i, carry):
            chunk = base + s + NS * i

            @pl.when(chunk < base + CCORE_CHUNKS)
            def _():
                off = pl.multiple_of(chunk * CCH, CCH)
                pltpu.sync_copy(srcd.at[0].at[pl.ds(off, CCH)], idxs)
                pltpu.async_copy(hn.at[idxs], rows, sem).wait()
                pltpu.sync_copy(srcd.at[1].at[pl.ds(off, CCH)], idxd)
                pltpu.sync_copy(rows, agg.at[idxd], add=True)
            return carry

        lax.fori_loop(0, CSUB_ITERS, loop, 0)
        plsc.subcore_barrier()
        for j in range(SEG_ITERS):
            k = s + NS * j

            @pl.when(k < NSEG)
            def _():
                pltpu.sync_copy(agg.at[pl.ds(k * DSEG, DSEG)], zv)
                pltpu.sync_copy(zv, out.at[c, p].at[pl.ds(k * DSEG, DSEG)])
        plsc.subcore_barrier()


def _sc_conv(hn, srcd, zeros16):
    return pl.kernel(
        _conv_body,
        out_type=jax.ShapeDtypeStruct((NC, NP, N, CW), f32),
        mesh=_mesh(),
        compiler_params=_SC_PARAMS,
        scratch_types=[
            pltpu.VMEM((CCH,), jnp.int32),
            pltpu.VMEM((CCH,), jnp.int32),
            pltpu.VMEM((CCH, CW), f32),
            pltpu.VMEM((DSEG, CW), f32),
            pltpu.VMEM_SHARED((N, CW), f32),
            pltpu.SemaphoreType.DMA,
        ],
    )(hn[0], hn[1], hn[2], hn[3], srcd, zeros16)


def _cls_body(a_tab, b_tab, ss, sd, out, ia, ib, bufa, bufb, bufo, sem):
    c = lax.axis_index("c")
    s = lax.axis_index("s")
    w = s * NC + c

    def loop(i, carry):
        chunk = w + NC * NS * i

        @pl.when(chunk < ECHUNKS)
        def _():
            off = pl.multiple_of(chunk * CH, CH)
            pltpu.sync_copy(ss.at[pl.ds(off, CH)], ia)
            pltpu.async_copy(a_tab.at[ia], bufa, sem).wait()
            pltpu.sync_copy(sd.at[pl.ds(off, CH)], ib)
            pltpu.async_copy(b_tab.at[ib], bufb, sem).wait()

            def radd(r, c2):
                for j in range(4):
                    bufo[r, pl.ds(16 * j, 16)] = (
                        bufa[r, pl.ds(16 * j, 16)] + bufb[r, pl.ds(16 * j, 16)])
                return c2

            lax.fori_loop(0, CH, radd, 0)
            pltpu.sync_copy(bufo, out.at[pl.ds(off, CH)])
        return carry

    lax.fori_loop(0, CLS_ITERS, loop, 0)


def _sc_edge_sum(a_tab, b_tab, ss, sd):
    return pl.kernel(
        _cls_body,
        out_type=jax.ShapeDtypeStruct((E, HID), f32),
        mesh=_mesh(),
        compiler_params=_SC_PARAMS,
        scratch_types=[
            pltpu.VMEM((CH,), jnp.int32),
            pltpu.VMEM((CH,), jnp.int32),
            pltpu.VMEM((CH, HID), f32),
            pltpu.VMEM((CH, HID), f32),
            pltpu.VMEM((CH, HID), f32),
            pltpu.SemaphoreType.DMA,
        ],
    )(a_tab, b_tab, ss, sd)


def _dot(a, b):
    return jnp.dot(a, b, preferred_element_type=f32,
                   precision=lax.Precision.HIGHEST)


def _dotd(a, b):
    return jnp.dot(a, b, preferred_element_type=f32)


def _leaky(x):
    return jnp.where(x >= 0, x, 0.01 * x)


def _nodes_body(s_ref, ids3_ref, tab_ref, ut_ref, fcw_ref, fcb_ref,
                emb3_ref, degp_ref, *out_and_scratch):
    hs_refs = out_and_scratch[0:NP]
    hu_refs = out_and_scratch[NP:2 * NP]
    xg_ref = out_and_scratch[2 * NP]
    ids = s_ref[...]
    iot = lax.broadcasted_iota(jnp.int32, (1, 128), 1)
    tab = tab_ref[...]
    for t in range(L):
        oh = (ids[:, t:t + 1] == iot).astype(f32)
        xg_ref[t] = _dot(oh, tab)
    ut = ut_ref[...]
    maskf = (iot // EMB) % 2 == 0
    h2 = jnp.zeros((BN, 2 * EMB), f32)
    c2 = jnp.zeros((BN, 2 * EMB), f32)
    for t in range(L):
        gx = jnp.where(maskf, xg_ref[t], xg_ref[L - 1 - t])
        g = gx + _dotd(h2, ut)
        sg = jax.nn.sigmoid(g[:, :6 * EMB])
        tg = jnp.tanh(g[:, 6 * EMB:])
        c2 = sg[:, 2 * EMB:4 * EMB] * c2 + sg[:, :2 * EMB] * tg
        h2 = sg[:, 4 * EMB:6 * EMB] * jnp.tanh(c2)
    h_url = _leaky(_dotd(h2, fcw_ref[...]) + fcb_ref[...])
    ids3 = ids3_ref[...]
    hc = _dot((ids3[:, 0:1] == iot).astype(f32), emb3_ref[0])
    hco = _dot((ids3[:, 1:2] == iot).astype(f32), emb3_ref[1])
    hsl = _dot((ids3[:, 2:3] == iot).astype(f32), emb3_ref[2])
    h = jnp.concatenate([h_url, hc, hco, hsl], 1)
    degp = degp_ref[...]

    def rfac(a):
        d = degp[0, a, :, 0:1] + degp[1, a, :, 0:1]
        return lax.rsqrt(jnp.clip(d, 1.0))

    hn_s = h * rfac(0)
    hn_u = h * rfac(2)
    for q in range(NP):
        hs_refs[q][...] = hn_s[:, CW * q:CW * (q + 1)]
        hu_refs[q][...] = hn_u[:, CW * q:CW * (q + 1)]


def _tc_nodes(inputs_s, ids3, tab, ut, fcw, fcb, emb3, degp):
    g = N // BN
    quarter = jax.ShapeDtypeStruct((N, CW), f32)
    outs = pl.pallas_call(
        _nodes_body,
        grid=(g,),
        in_specs=[
            pl.BlockSpec((BN, L), lambda i: (i, 0)),
            pl.BlockSpec((BN, 4), lambda i: (i, 0)),
            pl.BlockSpec((128, 128), lambda i: (0, 0)),
            pl.BlockSpec((2 * EMB, 8 * EMB), lambda i: (0, 0)),
            pl.BlockSpec((2 * EMB, EMB), lambda i: (0, 0)),
            pl.BlockSpec((1, EMB), lambda i: (0, 0)),
            pl.BlockSpec((3, 128, EMB), lambda i: (0, 0, 0)),
            pl.BlockSpec((NC, 4, BN, 8), lambda i: (0, 0, i, 0)),
        ],
        out_specs=[pl.BlockSpec((BN, CW), lambda i: (i, 0))] * (2 * NP),
        out_shape=[quarter] * (2 * NP),
        scratch_shapes=[pltpu.VMEM((L, BN, 128), f32)],
    )(inputs_s, ids3, tab, ut, fcw, fcb, emb3, degp)
    return outs[0:NP], outs[NP:2 * NP]


def _combine_core(ps, pu, degp, ws, bs, wu, bu):
    def agg(pr, a):
        x = jnp.concatenate([pr[0, q] + pr[1, q] for q in range(NP)], 1)
        d = degp[0, a, :, 0:1] + degp[1, a, :, 0:1]
        return x * lax.rsqrt(jnp.clip(d, 1.0))

    hs = agg(ps, 1)
    hu = agg(pu, 3)
    return _leaky(_dotd(hs, ws) + bs + _dotd(hu, wu) + bu)


def _combine0_body(ps_ref, pu_ref, degp_ref, ws_ref, bs_ref, wu_ref, bu_ref,
                   *out_refs):
    degp = degp_ref[...]
    h = _combine_core(ps_ref[...], pu_ref[...], degp, ws_ref[...], bs_ref[...],
                      wu_ref[...], bu_ref[...])

    def rfac(a):
        d = degp[0, a, :, 0:1] + degp[1, a, :, 0:1]
        return lax.rsqrt(jnp.clip(d, 1.0))

    hn_s = h * rfac(0)
    hn_u = h * rfac(2)
    for q in range(NP):
        out_refs[q][...] = hn_s[:, CW * q:CW * (q + 1)]
        out_refs[NP + q][...] = hn_u[:, CW * q:CW * (q + 1)]


def _combine1_body(ps_ref, pu_ref, degp_ref, ws_ref, bs_ref, wu_ref, bu_ref,
                   w1a_ref, w1b_ref, kvec_ref, a_ref, b_ref):
    h = _combine_core(ps_ref[...], pu_ref[...], degp_ref[...], ws_ref[...],
                      bs_ref[...], wu_ref[...], bu_ref[...])
    a_ref[...] = _dotd(h, w1a_ref[...]) + kvec_ref[...]
    b_ref[...] = _dotd(h, w1b_ref[...])


_SPEC_PART = pl.BlockSpec((NC, NP, BN, CW), lambda i: (0, 0, i, 0))
_SPEC_DEGP = pl.BlockSpec((NC, 4, BN, 8), lambda i: (0, 0, i, 0))
_SPEC_W = pl.BlockSpec((HID, HID), lambda i: (0, 0))
_SPEC_B = pl.BlockSpec((1, HID), lambda i: (0, 0))


def _tc_combine0(ps, pu, degp, ws, bs, wu, bu):
    g = N // BN
    quarter = jax.ShapeDtypeStruct((N, CW), f32)
    outs = pl.pallas_call(
        _combine0_body,
        grid=(g,),
        in_specs=[_SPEC_PART, _SPEC_PART, _SPEC_DEGP,
                  _SPEC_W, _SPEC_B, _SPEC_W, _SPEC_B],
        out_specs=[pl.BlockSpec((BN, CW), lambda i: (i, 0))] * (2 * NP),
        out_shape=[quarter] * (2 * NP),
    )(ps, pu, degp, ws, bs, wu, bu)
    return outs[0:NP], outs[NP:2 * NP]


def _tc_combine1(ps, pu, degp, ws, bs, wu, bu, w1a, w1b, kvec):
    g = N // BN
    full = jax.ShapeDtypeStruct((N, HID), f32)
    return pl.pallas_call(
        _combine1_body,
        grid=(g,),
        in_specs=[_SPEC_PART, _SPEC_PART, _SPEC_DEGP,
                  _SPEC_W, _SPEC_B, _SPEC_W, _SPEC_B,
                  _SPEC_W, _SPEC_W, _SPEC_B],
        out_specs=[pl.BlockSpec((BN, HID), lambda i: (i, 0))] * 2,
        out_shape=[full, full],
    )(ps, pu, degp, ws, bs, wu, bu, w1a, w1b, kvec)


def _logits_body(s_ref, w2_ref, b2_ref, o_ref):
    x = jnp.maximum(s_ref[...], 0.0)
    o_ref[...] = _dotd(x, w2_ref[...]) + b2_ref[...]


def _tc_logits(s_arr, w2p, b2p):
    g = E // BE
    return pl.pallas_call(
        _logits_body,
        grid=(g,),
        in_specs=[
            pl.BlockSpec((BE, HID), lambda i: (i, 0)),
            pl.BlockSpec((HID, 8), lambda i: (0, 0)),
            pl.BlockSpec((1, 8), lambda i: (0, 0)),
        ],
        out_specs=pl.BlockSpec((BE, 8), lambda i: (i, 0)),
        out_shape=jax.ShapeDtypeStruct((E, 8), f32),
    )(s_arr, w2p, b2p)


def kernel(inputs_s, inputs_sm, inputs_c, inputs_co, inputs_sl,
           edge_index_sim, edge_index_user, edge_sub_src, edge_sub_dst, params):
    p = params

    tab_f = p['emb_url'] @ p['w_ih_f'].T + p['b_ih_f'] + p['b_hh_f']
    tab_b = p['emb_url'] @ p['w_ih_b'].T + p['b_ih_b'] + p['b_hh_b']

    def gperm(tf, tb):
        gi, gf_, gg, go = (slice(0, 16), slice(16, 32), slice(32, 48),
                           slice(48, 64))
        return jnp.concatenate([tf[:, gi], tb[:, gi], tf[:, gf_], tb[:, gf_],
                                tf[:, go], tb[:, go], tf[:, gg], tb[:, gg]], 1)

    tab = gperm(tab_f, tab_b).astype(f32)
    utf = p['w_hh_f'].T
    utb = p['w_hh_b'].T
    ut = jnp.concatenate([
        gperm(utf, jnp.zeros_like(utf)),
        gperm(jnp.zeros_like(utb), utb)], 0).astype(f32)
    fcw = p['fc_w'].astype(f32)
    fcb = p['fc_b'].reshape(1, EMB).astype(f32)

    def pad128(t):
        return jnp.zeros((128, EMB), f32).at[:t.shape[0]].set(t)

    emb3 = jnp.stack([pad128(p['emb_cat']), pad128(p['emb_country']),
                      pad128(p['emb_sl'])])

    sbn = p['bn_g'] * lax.rsqrt(p['bn_v'] + 1e-5)
    w1a = (p['cls_w1'][:HID] * sbn[None, :]).astype(f32)
    w1b = (p['cls_w1'][HID:] * sbn[None, :]).astype(f32)
    kvec = ((p['cls_b1'] - p['bn_m']) * sbn + p['bn_b']).reshape(1, HID).astype(f32)
    w2p = jnp.zeros((HID, 8), f32).at[:, :2].set(p['cls_w2'])
    b2p = jnp.zeros((1, 8), f32).at[:, :2].set(p['cls_b2'])

    ids3 = jnp.concatenate([inputs_c, inputs_co, inputs_sl, inputs_sl], axis=1)
    idx4 = jnp.stack([edge_index_sim[0], edge_index_sim[1],
                      edge_index_user[0], edge_index_user[1]])
    zeros8 = jnp.zeros((DSEG, 8), f32)
    ones8 = jnp.ones((CH, 8), f32)
    zeros16 = jnp.zeros((DSEG, CW), f32)

    degp = _sc_degrees(idx4, zeros8, ones8)
    hs, hu = _tc_nodes(inputs_s, ids3, tab, ut, fcw, fcb, emb3, degp)

    ps = _sc_conv(hs, edge_index_sim, zeros16)
    pu = _sc_conv(hu, edge_index_user, zeros16)
    hs, hu = _tc_combine0(
        ps, pu, degp,
        p['gc0_sim_w'].astype(f32), p['gc0_sim_b'].reshape(1, HID).astype(f32),
        p['gc0_user_w'].astype(f32), p['gc0_user_b'].reshape(1, HID).astype(f32))

    ps = _sc_conv(hs, edge_index_sim, zeros16)
    pu = _sc_conv(hu, edge_index_user, zeros16)
    a_tab, b_tab = _tc_combine1(
        ps, pu, degp,
        p['gc1_sim_w'].astype(f32), p['gc1_sim_b'].reshape(1, HID).astype(f32),
        p['gc1_user_w'].astype(f32), p['gc1_user_b'].reshape(1, HID).astype(f32),
        w1a, w1b, kvec)

    s_arr = _sc_edge_sum(a_tab, b_tab, edge_sub_src, edge_sub_dst)
    out8 = _tc_logits(s_arr, w2p, b2p)
    return out8[:, :2]
